# Initial kernel scaffold; baseline (speedup 1.0000x reference)
import jax
import jax.numpy as jnp
from jax.experimental import pallas as pl


def kernel(features, edge_index, freqs, plr_w, plr_b, proj_w, proj_b, pre_bn_g, pre_bn_b, pre_w, pre_b, enc_bn_g, enc_bn_b, enc_ws, enc_wn, enc_bias, pred0_bn_g, pred0_bn_b, pred0_w, pred0_b, pred1_bn_g, pred1_bn_b, pred1_w, pred1_b):
    raise NotImplementedError("write your pallas kernel here")


# speedup vs baseline: 1.4245x; 1.4245x over previous
import functools
import math

import jax
import jax.numpy as jnp
from jax import lax
from jax.experimental import pallas as pl
from jax.experimental.pallas import tpu as pltpu
from jax.experimental.pallas import tpu_sc as plsc

_N = 10000
_E = 320000
_H = 128
_NF = 48
_DE = 16
_DEXT = 128

_NC = 2
_NS = 16
_NW = _NC * _NS
_EPW = _E // _NW
_CH = 80
_NCH = _EPW // _CH
_NP = 10240
_RPT = _NP // _NS
_RC = 80
_NRC = _RPT // _RC

_TWO_PI = 2.0 * math.pi
_INV_SQRT2 = 0.7071067811865476


def _gelu(x):
    return 0.5 * x * (1.0 + lax.erf(x * _INV_SQRT2))


def _col_stats(h):
    s0 = jnp.sum(h, axis=0, keepdims=True)
    s1 = jnp.sum(h * h, axis=0, keepdims=True)
    return jnp.concatenate([s0, s1, jnp.zeros((6, _H), jnp.float32)], axis=0)


def _bn_scale_shift(st, g, b):
    m = st[0:1, :] * (1.0 / _N)
    v = st[1:2, :] * (1.0 / _N) - m * m
    scale = g * lax.rsqrt(v + 1e-5)
    shift = b - m * scale
    return scale, shift


_FB = 400
_NFB = _N // _FB


def _front1_body(feat, freqs, plrw, plrb, emb_out):
    x = feat[...]
    f = freqs[...]
    xf = _TWO_PI * x[:, :, None] * f[None, :, :]
    c = jnp.cos(xf).reshape(_FB * _H, _NF)
    s = jnp.sin(xf).reshape(_FB * _H, _NF)
    w = plrw[...]
    emb_out[...] = jnp.maximum(c @ w[:_NF, :] + s @ w[_NF:, :] + plrb[...],
                               0.0)


_FRONT1_KW = dict(
    grid=(_NFB,),
    in_specs=[
        pl.BlockSpec((_FB, _H), lambda i: (i, 0)),
        pl.BlockSpec((_H, _NF), lambda i: (0, 0)),
        pl.BlockSpec((2 * _NF, _DE), lambda i: (0, 0)),
        pl.BlockSpec((1, _DE), lambda i: (0, 0)),
    ],
    out_specs=pl.BlockSpec((_FB * _H, _DE), lambda i: (i, 0)),
    out_shape=jax.ShapeDtypeStruct((_N * _H, _DE), jnp.float32),
)
_front1 = pl.pallas_call(_front1_body, **_FRONT1_KW)

_FB2 = 1000
_NFB2 = _N // _FB2


def _front2_body(emb, projw, projb, hraw, stats):
    i = pl.program_id(0)
    h = jnp.maximum(emb[...] @ projw[...] + projb[...], 0.0)
    hraw[...] = h
    st = _col_stats(h)

    @pl.when(i == 0)
    def _():
        stats[...] = st

    @pl.when(i > 0)
    def _():
        stats[...] += st


_FRONT2_KW = dict(
    grid=(_NFB2,),
    in_specs=[
        pl.BlockSpec((_FB2, _H * _DE), lambda i: (i, 0)),
        pl.BlockSpec((_H * _DE, _H), lambda i: (0, 0)),
        pl.BlockSpec((1, _H), lambda i: (0, 0)),
    ],
    out_specs=[
        pl.BlockSpec((_FB2, _H), lambda i: (i, 0)),
        pl.BlockSpec((8, _H), lambda i: (0, 0)),
    ],
    out_shape=[
        jax.ShapeDtypeStruct((_N, _H), jnp.float32),
        jax.ShapeDtypeStruct((8, _H), jnp.float32),
    ],
)
_front2 = pl.pallas_call(_front2_body, **_FRONT2_KW)


def _pre_body(hraw, st, g, b, w, bias, h_out, st_out):
    scale, shift = _bn_scale_shift(st[...], g[...], b[...])
    hn = hraw[...] * scale + shift
    h = _gelu(hn @ w[...] + bias[...])
    h_out[...] = h
    st_out[...] = _col_stats(h)


_pre = pl.pallas_call(
    _pre_body,
    out_shape=[
        jax.ShapeDtypeStruct((_N, _H), jnp.float32),
        jax.ShapeDtypeStruct((8, _H), jnp.float32),
    ],
)


def _enc_a_body(h, st, g, b, ws, hn_out, self_out):
    scale, shift = _bn_scale_shift(st[...], g[...], b[...])
    hn = h[...] * scale + shift
    hn_out[...] = hn
    self_out[...] = hn @ ws[...]


_enc_a = pl.pallas_call(
    _enc_a_body,
    out_shape=[
        jax.ShapeDtypeStruct((_N, _H), jnp.float32),
        jax.ShapeDtypeStruct((_N, _H), jnp.float32),
    ],
)


def _invdeg_body(parts, invd_out):
    deg = parts[0, :_N, :] + parts[1, :_N, :]
    invd_out[...] = 1.0 / jnp.maximum(deg, 1.0)


_invdeg = pl.pallas_call(
    _invdeg_body,
    out_shape=jax.ShapeDtypeStruct((_N, _H), jnp.float32),
)


def _enc_b_body(parts, invd, hn_in, selfm, wn, bias, h_out, st_out):
    agg = parts[0, :_N, :] + parts[1, :_N, :]
    hn = hn_in[...]
    conv = selfm[...] + (agg * invd[...]) @ wn[...] + bias[...]
    h = _gelu(conv + hn)
    h_out[...] = h
    st_out[...] = _col_stats(h)


_enc_b = pl.pallas_call(
    _enc_b_body,
    out_shape=[
        jax.ShapeDtypeStruct((_N, _H), jnp.float32),
        jax.ShapeDtypeStruct((8, _H), jnp.float32),
    ],
)


def _pred_a_body(h, st, g, b, w, bias, h_out, st_out):
    scale, shift = _bn_scale_shift(st[...], g[...], b[...])
    hn = h[...] * scale + shift
    h2 = hn + _gelu(hn @ w[...] + bias[...])
    h_out[...] = h2
    st_out[...] = _col_stats(h2)


_pred_a = pl.pallas_call(
    _pred_a_body,
    out_shape=[
        jax.ShapeDtypeStruct((_N, _H), jnp.float32),
        jax.ShapeDtypeStruct((8, _H), jnp.float32),
    ],
)


def _pred_b_body(h, st, g, b, w, bias, out):
    scale, shift = _bn_scale_shift(st[...], g[...], b[...])
    hn = h[...] * scale + shift
    wv = w[...][:, 0][None, :]
    out[...] = jnp.sum(hn * wv, axis=1, keepdims=True) + bias[...]


_pred_b = pl.pallas_call(
    _pred_b_body,
    out_shape=jax.ShapeDtypeStruct((_N, 1), jnp.float32),
)


def _sc_agg_body(x_hbm, src_hbm, dst_hbm, parts_hbm,
                 src_l, dst_l, rows, stage, agg_sh, sem):
    cid = lax.axis_index("c")
    sid = lax.axis_index("s")

    z16 = jnp.zeros((16,), jnp.float32)

    def zrow(r, carry):
        for j in range(_DEXT // 16):
            stage[r, pl.ds(j * 16, 16)] = z16
        return carry

    lax.fori_loop(0, _RC, zrow, 0)
    base = sid * _RPT

    def zcopy(k, carry):
        pltpu.sync_copy(stage, agg_sh.at[pl.ds(base + k * _RC, _RC)])
        return carry

    lax.fori_loop(0, _NRC, zcopy, 0)
    plsc.subcore_barrier()

    def body(ci, carry):
        pltpu.sync_copy(src_hbm.at[cid, sid, ci], src_l)
        pltpu.sync_copy(dst_hbm.at[cid, sid, ci], dst_l)
        pltpu.async_copy(x_hbm.at[src_l], rows, sem).wait()
        pltpu.sync_copy(rows, agg_sh.at[dst_l], add=True)
        return carry

    lax.fori_loop(0, _NCH, body, 0)
    plsc.subcore_barrier()

    def wb(k, carry):
        sl = pl.ds(base + k * _RC, _RC)
        pltpu.sync_copy(agg_sh.at[sl], stage)
        pltpu.sync_copy(stage, parts_hbm.at[cid, sl])
        return carry

    lax.fori_loop(0, _NRC, wb, 0)


_sc_agg_cache = []


def _sc_agg(hx, src, dst):
    if not _sc_agg_cache:
        _sc_agg_cache.append(functools.partial(
            pl.kernel,
            out_type=jax.ShapeDtypeStruct((_NC, _NP, _DEXT), jnp.float32),
            mesh=plsc.VectorSubcoreMesh(
                core_axis_name="c", subcore_axis_name="s",
                num_cores=_NC, num_subcores=_NS),
            scratch_types=[
                pltpu.VMEM((_CH,), jnp.int32),
                pltpu.VMEM((_CH,), jnp.int32),
                pltpu.VMEM((_CH, _DEXT), jnp.float32),
                pltpu.VMEM((_RC, _DEXT), jnp.float32),
                pltpu.VMEM_SHARED((_NP, _DEXT), jnp.float32),
                pltpu.SemaphoreType.DMA,
            ],
        )(_sc_agg_body))
    return _sc_agg_cache[0](hx, src, dst)


def kernel(features, edge_index, freqs, plr_w, plr_b, proj_w, proj_b,
           pre_bn_g, pre_bn_b, pre_w, pre_b,
           enc_bn_g, enc_bn_b, enc_ws, enc_wn, enc_bias,
           pred0_bn_g, pred0_bn_b, pred0_w, pred0_b,
           pred1_bn_g, pred1_bn_b, pred1_w, pred1_b):
    row = lambda v: v.reshape(1, -1)
    src = edge_index[0].reshape(_NC, _NS, _NCH, _CH)
    dst = edge_index[1].reshape(_NC, _NS, _NCH, _CH)

    emb = _front1(features, freqs, plr_w, row(plr_b))
    hraw, st = _front2(emb.reshape(_N, _H * _DE), proj_w, row(proj_b))
    h, st = _pre(hraw, st, row(pre_bn_g), row(pre_bn_b), pre_w, row(pre_b))
    ones_mat = jnp.ones((_N, _H), jnp.float32)
    invd = _invdeg(_sc_agg(ones_mat, src, dst))
    for i in range(3):
        hn, selfm = _enc_a(h, st, row(enc_bn_g[i]), row(enc_bn_b[i]),
                           enc_ws[i])
        parts = _sc_agg(hn, src, dst)
        h, st = _enc_b(parts, invd, hn, selfm, enc_wn[i], row(enc_bias[i]))
    h4, st4 = _pred_a(h, st, row(pred0_bn_g), row(pred0_bn_b),
                      pred0_w, row(pred0_b))
    out = _pred_b(h4, st4, row(pred1_bn_g), row(pred1_bn_b),
                  pred1_w, row(pred1_b))
    return out
